# baseline (speedup 1.0000x reference)
import functools

import jax
import jax.numpy as jnp
from jax import lax
from jax.experimental import pallas as pl
from jax.experimental.pallas import tpu as pltpu
from jax.experimental.pallas import tpu_sc as plsc

B = 320000
D = 128
NC = 2
NS = 16
NW = NC * NS
BPW = B // NW
C = 160
NCHUNK = BPW // C
TAIL = BPW - NCHUNK * C
L = 16
NG = C // L


def _sc_kernel(team_hbm, expert_hbm, ti_hbm, ei_hbm, out_hbm,
               ti_all, ei_all, trows, erows, out_all, pbuf, st, se):
    wid = lax.axis_index("s") * NC + lax.axis_index("c")
    base0 = wid * BPW
    iota_sc = lax.broadcasted_iota(jnp.int32, (L,), 0) * L

    pltpu.sync_copy(ti_hbm.at[pl.ds(base0, BPW)], ti_all)
    pltpu.sync_copy(ei_hbm.at[pl.ds(base0, BPW)], ei_all)
    pltpu.async_copy(team_hbm.at[ti_all.at[pl.ds(0, C)]], trows.at[0], st)
    pltpu.async_copy(expert_hbm.at[ei_all.at[pl.ds(0, C)]], erows.at[0], se)

    def chunk_body(i, j):
        pltpu.make_async_copy(team_hbm.at[pl.ds(0, C)], trows.at[j], st).wait()
        pltpu.make_async_copy(expert_hbm.at[pl.ds(0, C)], erows.at[j], se).wait()

        @pl.when(i + 1 < NCHUNK)
        def _prefetch():
            nxt = (i + 1) * C
            pltpu.async_copy(team_hbm.at[ti_all.at[pl.ds(nxt, C)]],
                             trows.at[1 - j], st)
            pltpu.async_copy(expert_hbm.at[ei_all.at[pl.ds(nxt, C)]],
                             erows.at[1 - j], se)

        _dot_block(trows.at[j], erows.at[j], out_all, pbuf, iota_sc,
                   i * C, C)
        return 1 - j

    jf = lax.fori_loop(0, NCHUNK, chunk_body, 0)

    tb = NCHUNK * C
    pltpu.async_copy(team_hbm.at[ti_all.at[pl.ds(tb, TAIL)]],
                     trows.at[jf, pl.ds(0, TAIL)], st)
    pltpu.async_copy(expert_hbm.at[ei_all.at[pl.ds(tb, TAIL)]],
                     erows.at[jf, pl.ds(0, TAIL)], se)
    pltpu.make_async_copy(team_hbm.at[pl.ds(0, TAIL)],
                          trows.at[jf, pl.ds(0, TAIL)], st).wait()
    pltpu.make_async_copy(expert_hbm.at[pl.ds(0, TAIL)],
                          erows.at[jf, pl.ds(0, TAIL)], se).wait()
    _dot_block(trows.at[jf], erows.at[jf], out_all, pbuf, iota_sc, tb, TAIL)

    pltpu.sync_copy(out_all, out_hbm.at[pl.ds(base0, BPW)])


def _dot_block(tr, er, out_all, pbuf, iota_sc, obase, count):
    @plsc.parallel_loop(0, count, 1, unroll=8)
    def _edges(el):
        acc = tr[el, pl.ds(0, L)] * er[el, pl.ds(0, L)]
        for f in range(1, D // L):
            acc = acc + tr[el, pl.ds(f * L, L)] * er[el, pl.ds(f * L, L)]
        pbuf[pl.ds(el * L, L)] = acc

    for g in range(count // L):
        gbase = g * L * L
        tot = plsc.load_gather(pbuf, [iota_sc + gbase])
        for l in range(1, L):
            tot = tot + plsc.load_gather(pbuf, [iota_sc + (gbase + l)])
        out_all[pl.ds(obase + g * L, L)] = tot


@jax.jit
def _run(x_expert, x_team, team_idx, expert_idx):
    mesh = plsc.VectorSubcoreMesh(core_axis_name="c", subcore_axis_name="s")
    k = functools.partial(
        pl.kernel,
        out_type=jax.ShapeDtypeStruct((B,), jnp.float32),
        mesh=mesh,
        compiler_params=pltpu.CompilerParams(needs_layout_passes=False),
        scratch_types=[
            pltpu.VMEM((BPW,), jnp.int32),
            pltpu.VMEM((BPW,), jnp.int32),
            pltpu.VMEM((2, C, D), jnp.float32),
            pltpu.VMEM((2, C, D), jnp.float32),
            pltpu.VMEM((BPW,), jnp.float32),
            pltpu.VMEM((C * L,), jnp.float32),
            pltpu.SemaphoreType.DMA,
            pltpu.SemaphoreType.DMA,
        ],
    )(_sc_kernel)
    return k(x_team, x_expert, team_idx, expert_idx)


def kernel(x_expert, x_team, edge_label_index_team_experts):
    idx = edge_label_index_team_experts.astype(jnp.int32)
    return _run(x_expert, x_team, idx[0], idx[1])


# speedup vs baseline: 1.0088x; 1.0088x over previous
import functools

import jax
import jax.numpy as jnp
from jax import lax
from jax.experimental import pallas as pl
from jax.experimental.pallas import tpu as pltpu
from jax.experimental.pallas import tpu_sc as plsc

B = 320000
D = 128
NC = 2
NS = 16
NW = NC * NS
BPW = B // NW
C = 160
NCHUNK = BPW // C
TAIL = BPW - NCHUNK * C
L = 16
NG = C // L


def _sc_kernel(team_hbm, expert_hbm, ti_hbm, ei_hbm, out_hbm,
               ti_all, ei_all, trows, erows, out_all, pbuf, st, se):
    wid = lax.axis_index("s") * NC + lax.axis_index("c")
    base0 = wid * BPW
    iota_sc = lax.broadcasted_iota(jnp.int32, (L,), 0) * L

    pltpu.async_copy(ti_hbm.at[pl.ds(base0, BPW)], ti_all, st)
    pltpu.async_copy(ei_hbm.at[pl.ds(base0, BPW)], ei_all, se)
    pltpu.make_async_copy(ti_hbm.at[pl.ds(base0, BPW)], ti_all, st).wait()
    pltpu.make_async_copy(ei_hbm.at[pl.ds(base0, BPW)], ei_all, se).wait()
    pltpu.async_copy(team_hbm.at[ti_all.at[pl.ds(0, C)]], trows.at[0], st)
    pltpu.async_copy(expert_hbm.at[ei_all.at[pl.ds(0, C)]], erows.at[0], se)

    def chunk_body(i, j):
        pltpu.make_async_copy(team_hbm.at[pl.ds(0, C)], trows.at[j], st).wait()
        pltpu.make_async_copy(expert_hbm.at[pl.ds(0, C)], erows.at[j], se).wait()

        @pl.when(i + 1 < NCHUNK)
        def _prefetch():
            nxt = (i + 1) * C
            pltpu.async_copy(team_hbm.at[ti_all.at[pl.ds(nxt, C)]],
                             trows.at[1 - j], st)
            pltpu.async_copy(expert_hbm.at[ei_all.at[pl.ds(nxt, C)]],
                             erows.at[1 - j], se)

        @pl.when(i + 1 == NCHUNK)
        def _prefetch_tail():
            tb = NCHUNK * C
            pltpu.async_copy(team_hbm.at[ti_all.at[pl.ds(tb, TAIL)]],
                             trows.at[1 - j, pl.ds(0, TAIL)], st)
            pltpu.async_copy(expert_hbm.at[ei_all.at[pl.ds(tb, TAIL)]],
                             erows.at[1 - j, pl.ds(0, TAIL)], se)

        _dot_block(trows.at[j], erows.at[j], out_all, pbuf, iota_sc,
                   i * C, C)
        return 1 - j

    jf = lax.fori_loop(0, NCHUNK, chunk_body, 0)

    tb = NCHUNK * C
    pltpu.make_async_copy(team_hbm.at[pl.ds(0, TAIL)],
                          trows.at[jf, pl.ds(0, TAIL)], st).wait()
    pltpu.make_async_copy(expert_hbm.at[pl.ds(0, TAIL)],
                          erows.at[jf, pl.ds(0, TAIL)], se).wait()
    _dot_block(trows.at[jf], erows.at[jf], out_all, pbuf, iota_sc, tb, TAIL)

    pltpu.sync_copy(out_all, out_hbm.at[pl.ds(base0, BPW)])


def _dot_block(tr, er, out_all, pbuf, iota_sc, obase, count):
    @plsc.parallel_loop(0, count, 1, unroll=8)
    def _edges(el):
        acc = tr[el, pl.ds(0, L)] * er[el, pl.ds(0, L)]
        for f in range(1, D // L):
            acc = acc + tr[el, pl.ds(f * L, L)] * er[el, pl.ds(f * L, L)]
        pbuf[pl.ds(el * L, L)] = acc

    for g in range(count // L):
        gbase = g * L * L
        tot = plsc.load_gather(pbuf, [iota_sc + gbase])
        for l in range(1, L):
            tot = tot + plsc.load_gather(pbuf, [iota_sc + (gbase + l)])
        out_all[pl.ds(obase + g * L, L)] = tot


@jax.jit
def _run(x_expert, x_team, team_idx, expert_idx):
    mesh = plsc.VectorSubcoreMesh(core_axis_name="c", subcore_axis_name="s")
    k = functools.partial(
        pl.kernel,
        out_type=jax.ShapeDtypeStruct((B,), jnp.float32),
        mesh=mesh,
        compiler_params=pltpu.CompilerParams(needs_layout_passes=False),
        scratch_types=[
            pltpu.VMEM((BPW,), jnp.int32),
            pltpu.VMEM((BPW,), jnp.int32),
            pltpu.VMEM((2, C, D), jnp.float32),
            pltpu.VMEM((2, C, D), jnp.float32),
            pltpu.VMEM((BPW,), jnp.float32),
            pltpu.VMEM((C * L,), jnp.float32),
            pltpu.SemaphoreType.DMA,
            pltpu.SemaphoreType.DMA,
        ],
    )(_sc_kernel)
    return k(x_team, x_expert, team_idx, expert_idx)


def kernel(x_expert, x_team, edge_label_index_team_experts):
    idx = edge_label_index_team_experts.astype(jnp.int32)
    return _run(x_expert, x_team, idx[0], idx[1])


# speedup vs baseline: 1.0245x; 1.0156x over previous
import functools

import jax
import jax.numpy as jnp
from jax import lax
from jax.experimental import pallas as pl
from jax.experimental.pallas import tpu as pltpu
from jax.experimental.pallas import tpu_sc as plsc

B = 320000
D = 128
NC = 2
NS = 16
NW = NC * NS
BPW = B // NW
C = 176
NCHUNK = BPW // C
TAIL = BPW - NCHUNK * C
L = 16
NG = C // L


def _sc_kernel(team_hbm, expert_hbm, ti_hbm, ei_hbm, out_hbm,
               ti_all, ei_all, trows, erows, out_all, pbuf, st, se):
    wid = lax.axis_index("s") * NC + lax.axis_index("c")
    base0 = wid * BPW
    iota_sc = lax.broadcasted_iota(jnp.int32, (L,), 0) * L

    pltpu.async_copy(ti_hbm.at[pl.ds(base0, BPW)], ti_all, st)
    pltpu.async_copy(ei_hbm.at[pl.ds(base0, BPW)], ei_all, se)
    pltpu.make_async_copy(ti_hbm.at[pl.ds(base0, BPW)], ti_all, st).wait()
    pltpu.make_async_copy(ei_hbm.at[pl.ds(base0, BPW)], ei_all, se).wait()
    pltpu.async_copy(team_hbm.at[ti_all.at[pl.ds(0, C)]], trows.at[0], st)
    pltpu.async_copy(expert_hbm.at[ei_all.at[pl.ds(0, C)]], erows.at[0], se)

    def chunk_body(i, j):
        pltpu.make_async_copy(team_hbm.at[pl.ds(0, C)], trows.at[j], st).wait()
        pltpu.make_async_copy(expert_hbm.at[pl.ds(0, C)], erows.at[j], se).wait()

        @pl.when(i + 1 < NCHUNK)
        def _prefetch():
            nxt = (i + 1) * C
            pltpu.async_copy(team_hbm.at[ti_all.at[pl.ds(nxt, C)]],
                             trows.at[1 - j], st)
            pltpu.async_copy(expert_hbm.at[ei_all.at[pl.ds(nxt, C)]],
                             erows.at[1 - j], se)

        @pl.when(i + 1 == NCHUNK)
        def _prefetch_tail():
            tb = NCHUNK * C
            pltpu.async_copy(team_hbm.at[ti_all.at[pl.ds(tb, TAIL)]],
                             trows.at[1 - j, pl.ds(0, TAIL)], st)
            pltpu.async_copy(expert_hbm.at[ei_all.at[pl.ds(tb, TAIL)]],
                             erows.at[1 - j, pl.ds(0, TAIL)], se)

        _dot_block(trows.at[j], erows.at[j], out_all, pbuf, iota_sc,
                   i * C, C)
        return 1 - j

    jf = lax.fori_loop(0, NCHUNK, chunk_body, 0)

    tb = NCHUNK * C
    pltpu.make_async_copy(team_hbm.at[pl.ds(0, TAIL)],
                          trows.at[jf, pl.ds(0, TAIL)], st).wait()
    pltpu.make_async_copy(expert_hbm.at[pl.ds(0, TAIL)],
                          erows.at[jf, pl.ds(0, TAIL)], se).wait()
    _dot_block(trows.at[jf], erows.at[jf], out_all, pbuf, iota_sc, tb, TAIL)

    pltpu.sync_copy(out_all, out_hbm.at[pl.ds(base0, BPW)])


def _dot_block(tr, er, out_all, pbuf, iota_sc, obase, count):
    @plsc.parallel_loop(0, count, 1, unroll=8)
    def _edges(el):
        acc = tr[el, pl.ds(0, L)] * er[el, pl.ds(0, L)]
        for f in range(1, D // L):
            acc = acc + tr[el, pl.ds(f * L, L)] * er[el, pl.ds(f * L, L)]
        pbuf[pl.ds(el * L, L)] = acc

    for g in range(count // L):
        gbase = g * L * L
        tot = plsc.load_gather(pbuf, [iota_sc + gbase])
        for l in range(1, L):
            tot = tot + plsc.load_gather(pbuf, [iota_sc + (gbase + l)])
        out_all[pl.ds(obase + g * L, L)] = tot


@jax.jit
def _run(x_expert, x_team, team_idx, expert_idx):
    mesh = plsc.VectorSubcoreMesh(core_axis_name="c", subcore_axis_name="s")
    k = functools.partial(
        pl.kernel,
        out_type=jax.ShapeDtypeStruct((B,), jnp.float32),
        mesh=mesh,
        compiler_params=pltpu.CompilerParams(needs_layout_passes=False),
        scratch_types=[
            pltpu.VMEM((BPW,), jnp.int32),
            pltpu.VMEM((BPW,), jnp.int32),
            pltpu.VMEM((2, C, D), jnp.float32),
            pltpu.VMEM((2, C, D), jnp.float32),
            pltpu.VMEM((BPW,), jnp.float32),
            pltpu.VMEM((C * L,), jnp.float32),
            pltpu.SemaphoreType.DMA,
            pltpu.SemaphoreType.DMA,
        ],
    )(_sc_kernel)
    return k(x_team, x_expert, team_idx, expert_idx)


def kernel(x_expert, x_team, edge_label_index_team_experts):
    idx = edge_label_index_team_experts.astype(jnp.int32)
    return _run(x_expert, x_team, idx[0], idx[1])


# speedup vs baseline: 1.0486x; 1.0235x over previous
import functools

import jax
import jax.numpy as jnp
from jax import lax
from jax.experimental import pallas as pl
from jax.experimental.pallas import tpu as pltpu
from jax.experimental.pallas import tpu_sc as plsc

B = 320000
D = 128
NC = 2
NS = 16
NW = NC * NS
BPW = B // NW
C = 200
NCHUNK = BPW // C
L = 16


def _sc_kernel(team_hbm, expert_hbm, ti_hbm, ei_hbm, out_hbm,
               ti_all, ei_all, trows, erows, obuf, pbuf, st, se, so):

    wid = lax.axis_index("s") * NC + lax.axis_index("c")
    base0 = wid * BPW
    iota_sc = lax.broadcasted_iota(jnp.int32, (L,), 0) * L

    pltpu.async_copy(ti_hbm.at[pl.ds(base0, BPW)], ti_all, st)
    pltpu.async_copy(ei_hbm.at[pl.ds(base0, BPW)], ei_all, se)
    pltpu.make_async_copy(ti_hbm.at[pl.ds(base0, BPW)], ti_all, st).wait()
    pltpu.make_async_copy(ei_hbm.at[pl.ds(base0, BPW)], ei_all, se).wait()
    pltpu.async_copy(team_hbm.at[ti_all.at[pl.ds(0, C)]], trows.at[0], st)
    pltpu.async_copy(expert_hbm.at[ei_all.at[pl.ds(0, C)]], erows.at[0], se)

    def chunk_body(i, j):
        pltpu.make_async_copy(team_hbm.at[pl.ds(0, C)], trows.at[j], st).wait()
        pltpu.make_async_copy(expert_hbm.at[pl.ds(0, C)], erows.at[j], se).wait()

        @pl.when(i + 1 < NCHUNK)
        def _prefetch():
            nxt = (i + 1) * C
            pltpu.async_copy(team_hbm.at[ti_all.at[pl.ds(nxt, C)]],
                             trows.at[1 - j], st)
            pltpu.async_copy(expert_hbm.at[ei_all.at[pl.ds(nxt, C)]],
                             erows.at[1 - j], se)

        _edge_dots(trows.at[j], erows.at[j], pbuf)

        @pl.when(i >= 1)
        def _drain_out():
            pltpu.make_async_copy(obuf, out_hbm.at[pl.ds(0, C)], so).wait()

        _group_sums(pbuf, obuf, iota_sc)
        pltpu.async_copy(obuf, out_hbm.at[pl.ds(base0 + i * C, C)], so)
        return 1 - j

    lax.fori_loop(0, NCHUNK, chunk_body, 0)
    pltpu.make_async_copy(obuf, out_hbm.at[pl.ds(0, C)], so).wait()


def _edge_dots(tr, er, pbuf):
    @plsc.parallel_loop(0, C, 1, unroll=8)
    def _edges(el):
        acc = tr[el, pl.ds(0, L)] * er[el, pl.ds(0, L)]
        for f in range(1, D // L):
            acc = acc + tr[el, pl.ds(f * L, L)] * er[el, pl.ds(f * L, L)]
        pbuf[pl.ds(el * L, L)] = acc


def _group_sums(pbuf, oref, iota_sc):
    starts = list(range(0, C - L + 1, L))
    if C % L:
        starts.append(C - L)
    for s in starts:
        gbase = s * L
        tot = plsc.load_gather(pbuf, [iota_sc + gbase])
        for l in range(1, L):
            tot = tot + plsc.load_gather(pbuf, [iota_sc + (gbase + l)])
        oref[pl.ds(s, L)] = tot


@jax.jit
def _run(x_expert, x_team, team_idx, expert_idx):
    mesh = plsc.VectorSubcoreMesh(core_axis_name="c", subcore_axis_name="s")
    k = functools.partial(
        pl.kernel,
        out_type=jax.ShapeDtypeStruct((B,), jnp.float32),
        mesh=mesh,
        compiler_params=pltpu.CompilerParams(needs_layout_passes=False),
        scratch_types=[
            pltpu.VMEM((BPW,), jnp.int32),
            pltpu.VMEM((BPW,), jnp.int32),
            pltpu.VMEM((2, C, D), jnp.float32),
            pltpu.VMEM((2, C, D), jnp.float32),
            pltpu.VMEM((C,), jnp.float32),
            pltpu.VMEM((C * L,), jnp.float32),
            pltpu.SemaphoreType.DMA,
            pltpu.SemaphoreType.DMA,
            pltpu.SemaphoreType.DMA,
        ],
    )(_sc_kernel)
    return k(x_team, x_expert, team_idx, expert_idx)


def kernel(x_expert, x_team, edge_label_index_team_experts):
    idx = edge_label_index_team_experts.astype(jnp.int32)
    return _run(x_expert, x_team, idx[0], idx[1])
